# Initial kernel scaffold; baseline (speedup 1.0000x reference)
import jax
import jax.numpy as jnp
from jax.experimental import pallas as pl


def kernel(x0, x1, x2, Ws0, Ws1, Ws2, Wu0, Wu1, Wb1, Wb2, b0, b1, b2, Wh1, bh1, Wh2, bh2, boundary_index_1, boundary_index_2, upper_adj_index_0, upper_adj_index_1, batch0, batch1, batch2):
    raise NotImplementedError("write your pallas kernel here")


# speedup vs baseline: 1.1632x; 1.1632x over previous
import functools

import jax
import jax.numpy as jnp
from jax import lax
from jax.experimental import pallas as pl
from jax.experimental.pallas import tpu as pltpu
from jax.experimental.pallas import tpu_sc as plsc

D = 128
NC = 2
NS = 16
L = 16
R = 13312
CH = 128


def _mesh():
    return plsc.VectorSubcoreMesh(
        core_axis_name="c", subcore_axis_name="s", num_cores=NC, num_subcores=NS
    )


def _cdiv(a, b):
    return (a + b - 1) // b


def _seg_sum_sc(h, src, dst, n_out):
    E = src.shape[0]
    npass = _cdiv(n_out, NC * R)
    n_pad = npass * NC * R
    nch = _cdiv(E, CH)
    rpt = R // NS

    @functools.partial(
        pl.kernel,
        out_type=jax.ShapeDtypeStruct((n_pad, D), jnp.float32),
        mesh=_mesh(),
        scratch_types=[
            pltpu.VMEM_SHARED((R + 8, D), jnp.float32),
            pltpu.VMEM((CH, D), jnp.float32),
            pltpu.VMEM((1, CH), jnp.int32),
            pltpu.VMEM((1, CH), jnp.int32),
            pltpu.VMEM((CH,), jnp.int32),
            pltpu.VMEM((32, D), jnp.float32),
            pltpu.SemaphoreType.DMA,
        ],
    )
    def body(h_hbm, src_hbm, dst_hbm, out_hbm, acc, rows, gidx, sidxr,
             dstage, zbuf, sem):
        cid = lax.axis_index("c")
        sid = lax.axis_index("s")
        zero16 = jnp.zeros((L,), jnp.float32)
        iot = lax.iota(jnp.int32, L)

        def zrow(r, _):
            for k in range(D // L):
                zbuf[r, pl.ds(k * L, L)] = zero16
            return 0
        lax.fori_loop(0, 32, zrow, 0)

        for p in range(npass):
            base = (p * NC + cid) * R
            for c0 in range(0, rpt, 32):
                csz = min(32, rpt - c0)
                pltpu.sync_copy(zbuf.at[pl.ds(0, csz)],
                                acc.at[pl.ds(sid * rpt + c0, csz)])
            plsc.subcore_barrier()

            def chunk_body(k, _):
                c = sid + k * NS
                e0 = c * CH
                ew = jnp.minimum(e0, E - CH)
                pltpu.sync_copy(src_hbm.at[pl.ds(ew, CH)], gidx.at[0])
                cp = pltpu.async_copy(h_hbm.at[gidx.at[0]], rows, sem)
                pltpu.sync_copy(dst_hbm.at[pl.ds(ew, CH)], dstage)
                for j in range(CH // L):
                    dv = dstage[pl.ds(j * L, L)]
                    dl = dv - base
                    epos = ew + j * L + iot
                    ok = (dl >= 0) & (dl < R) & (epos >= e0)
                    sidxr[0, pl.ds(j * L, L)] = jnp.where(ok, dl, jnp.int32(R))
                cp.wait()
                pltpu.sync_copy(rows, acc.at[sidxr.at[0]], add=True)
                return 0
            trips = jnp.maximum((nch - sid + NS - 1) // NS, 0)
            lax.fori_loop(0, trips, chunk_body, 0)

            plsc.subcore_barrier()
            for c0 in range(0, rpt, 128):
                csz = min(128, rpt - c0)
                lo = sid * rpt + c0
                pltpu.sync_copy(acc.at[pl.ds(lo, csz)],
                                out_hbm.at[pl.ds(p * NC * R + cid * R + lo, csz)])
            if p + 1 < npass:
                plsc.subcore_barrier()

    return body(h, src, dst)


def _pool_sc(h, batch):
    N = h.shape[0]
    G = 256
    assert N % (2 * 8) == 0
    nsc = N // NC
    assert nsc >= CH
    nch = _cdiv(nsc, CH)

    @functools.partial(
        pl.kernel,
        out_type=(jax.ShapeDtypeStruct((NC, G, D), jnp.float32),
                  jax.ShapeDtypeStruct((NC, G, D), jnp.float32)),
        mesh=_mesh(),
        scratch_types=[
            pltpu.VMEM_SHARED((G + 8, D), jnp.float32),
            pltpu.VMEM_SHARED((G + 8, D), jnp.float32),
            pltpu.VMEM((CH, D), jnp.float32),
            pltpu.VMEM((CH,), jnp.int32),
            pltpu.VMEM((1, CH), jnp.int32),
            pltpu.VMEM((CH, D), jnp.float32),
            pltpu.VMEM((16, D), jnp.float32),
            pltpu.SemaphoreType.DMA,
        ],
    )
    def body(h_hbm, b_hbm, sum_hbm, cnt_hbm, accs, accc, rows, bbuf, idxrow,
             obuf, zbuf, sem):
        cid = lax.axis_index("c")
        sid = lax.axis_index("s")
        base = cid * nsc
        one16 = jnp.ones((L,), jnp.float32)
        zero16 = jnp.zeros((L,), jnp.float32)

        def orow(r, _):
            for k in range(D // L):
                obuf[r, pl.ds(k * L, L)] = one16
            return 0
        lax.fori_loop(0, CH, orow, 0)

        def zrow(r, _):
            for k in range(D // L):
                zbuf[r, pl.ds(k * L, L)] = zero16
            return 0
        lax.fori_loop(0, 16, zrow, 0)
        gpt0 = G // NS
        pltpu.sync_copy(zbuf, accs.at[pl.ds(sid * gpt0, gpt0)])
        pltpu.sync_copy(zbuf, accc.at[pl.ds(sid * gpt0, gpt0)])
        plsc.subcore_barrier()

        def cbody(k, _):
            c = sid + k * NS

            @pl.when(c < nch)
            def _():
                ws = jnp.minimum(c * CH, nsc - CH)
                cp = pltpu.async_copy(h_hbm.at[pl.ds(base + ws, CH)], rows, sem)
                pltpu.sync_copy(b_hbm.at[pl.ds(base + ws, CH)], bbuf)
                iot = lax.iota(jnp.int32, L)
                for j in range(CH // L):
                    b_v = bbuf[pl.ds(j * L, L)]
                    gpos = ws + j * L + iot
                    b_v = jnp.where(gpos >= c * CH, b_v, jnp.int32(G))
                    idxrow[0, pl.ds(j * L, L)] = b_v
                cp.wait()
                pltpu.sync_copy(rows, accs.at[idxrow.at[0]], add=True)
                pltpu.sync_copy(obuf, accc.at[idxrow.at[0]], add=True)
            return 0
        lax.fori_loop(0, _cdiv(nch, NS), cbody, 0)
        plsc.subcore_barrier()
        gpt = G // NS
        pltpu.sync_copy(accs.at[pl.ds(sid * gpt, gpt)],
                        sum_hbm.at[cid, pl.ds(sid * gpt, gpt)])
        pltpu.sync_copy(accc.at[pl.ds(sid * gpt, gpt)],
                        cnt_hbm.at[cid, pl.ds(sid * gpt, gpt)])

    return body(h, batch)


BR = 2000


def _mm_relu(xs, ws, bias, n_rows, row_offs=None, gated=None):
    nblk = n_rows // BR
    assert n_rows % BR == 0
    if row_offs is None:
        row_offs = [0] * len(xs)
    if gated is None:
        gated = [None] * len(xs)

    def mk_spec(off, gate):
        def imap(i):
            j = i + off
            if gate is not None:
                j = jnp.minimum(j, gate - 1)
            return (j, 0)
        return pl.BlockSpec((BR, D), imap)

    in_specs = [mk_spec(o, g) for o, g in zip(row_offs, gated)]
    in_specs += [pl.BlockSpec((D, D), lambda i: (0, 0))] * len(ws)
    in_specs += [pl.BlockSpec((1, D), lambda i: (0, 0))]

    def body(*refs):
        i = pl.program_id(0)
        x_refs = refs[:len(xs)]
        w_refs = refs[len(xs):2 * len(xs)]
        b_ref = refs[2 * len(xs)]
        o_ref = refs[2 * len(xs) + 1]
        acc = jnp.dot(x_refs[0][...], w_refs[0][...],
                      preferred_element_type=jnp.float32)
        for t in range(1, len(xs)):
            term = jnp.dot(x_refs[t][...], w_refs[t][...],
                           preferred_element_type=jnp.float32)
            if gated[t] is not None:
                term = jnp.where(i < gated[t], term, 0.0)
            acc = acc + term
        acc = acc + b_ref[...]
        o_ref[...] = jnp.maximum(acc, 0.0)

    return pl.pallas_call(
        body,
        grid=(nblk,),
        in_specs=in_specs,
        out_specs=pl.BlockSpec((BR, D), lambda i: (i, 0)),
        out_shape=jax.ShapeDtypeStruct((n_rows, D), jnp.float32),
    )(*xs, *ws, bias)


def _final_mlp(parts, Wh1, bh1, Wh2, bh2):
    flat = [a for pr in parts for a in pr]

    def body(*refs):
        s0, c0, s1, c1, s2, c2 = refs[:6]
        wh1, b1r, wh2, b2r, o = refs[6:]
        g = jnp.zeros((256, D), jnp.float32)
        for s, c in ((s0, c0), (s1, c1), (s2, c2)):
            ssum = s[0] + s[1]
            cnt = jnp.maximum(c[0] + c[1], 1.0)
            g = g + ssum / cnt
        hid = jnp.maximum(
            jnp.dot(g, wh1[...], preferred_element_type=jnp.float32)
            + b1r[...], 0.0)
        o[...] = (jnp.dot(hid, wh2[...], preferred_element_type=jnp.float32)
                  + b2r[...])

    return pl.pallas_call(
        body,
        in_specs=[pl.BlockSpec((NC, 256, D), lambda: (0, 0, 0))] * 6
        + [pl.BlockSpec((D, D), lambda: (0, 0)),
           pl.BlockSpec((1, D), lambda: (0, 0)),
           pl.BlockSpec((D, 1), lambda: (0, 0)),
           pl.BlockSpec((1, 1), lambda: (0, 0))],
        out_specs=pl.BlockSpec((256, 1), lambda: (0, 0)),
        out_shape=jax.ShapeDtypeStruct((256, 1), jnp.float32),
    )(*flat, Wh1, bh1.reshape(1, D), Wh2, bh2.reshape(1, 1))


def kernel(x0, x1, x2, Ws0, Ws1, Ws2, Wu0, Wu1, Wb1, Wb2, b0, b1, b2, Wh1,
           bh1, Wh2, bh2, boundary_index_1, boundary_index_2,
           upper_adj_index_0, upper_adj_index_1, batch0, batch1, batch2):
    N0, N1, N2 = x0.shape[0], x1.shape[0], x2.shape[0]
    ua0s, ua0d = upper_adj_index_0[0], upper_adj_index_0[1]
    ua1s, ua1d = upper_adj_index_1[0], upper_adj_index_1[1]
    b1s, b1d = boundary_index_1[0], boundary_index_1[1]
    b2s, b2d = boundary_index_2[0], boundary_index_2[1]
    b0r, b1r, b2r = (b.reshape(1, D) for b in (b0, b1, b2))
    nblk0 = N0 // BR

    def layer(h0, h1, h2):
        mu0 = _seg_sum_sc(h0, ua0s, ua0d, N0)
        mb1 = _seg_sum_sc(h0, b1s, b1d, N0)
        mu1 = _seg_sum_sc(h1, ua1s, ua1d, N1)
        mb2 = _seg_sum_sc(h1, b2s, b2d, N2)
        n0 = _mm_relu([h0, mu0], [Ws0, Wu0], b0r, N0)
        n1 = _mm_relu([h1, mu1, mb1], [Ws1, Wu1, Wb1], b1r, N1,
                      gated=[None, None, nblk0])
        n2 = _mm_relu([h2, mb2], [Ws2, Wb2], b2r, N2)
        return n0, n1, n2

    h0, h1, h2 = layer(x0, x1, x2)
    h0, h1, h2 = layer(h0, h1, h2)
    parts = [_pool_sc(h0, batch0), _pool_sc(h1, batch1), _pool_sc(h2, batch2)]
    return _final_mlp(parts, Wh1, bh1, Wh2, bh2)


# speedup vs baseline: 1.4978x; 1.2876x over previous
import functools

import jax
import jax.numpy as jnp
from jax import lax
from jax.experimental import pallas as pl
from jax.experimental.pallas import tpu as pltpu
from jax.experimental.pallas import tpu_sc as plsc

D = 128
NC = 2
NS = 16
L = 16
R = 12544
CH = 112


def _mesh():
    return plsc.VectorSubcoreMesh(
        core_axis_name="c", subcore_axis_name="s", num_cores=NC, num_subcores=NS
    )


def _cdiv(a, b):
    return (a + b - 1) // b


def _seg_sum_sc(h, src, dst, zer, n_out):
    E = src.shape[0]
    npass = _cdiv(n_out, NC * R)
    n_pad = npass * NC * R
    nch = _cdiv(E, CH)
    rpt = R // NS
    trips = _cdiv(_cdiv(nch, NS), 2)

    @functools.partial(
        pl.kernel,
        out_type=jax.ShapeDtypeStruct((n_pad, D), jnp.float32),
        mesh=_mesh(),
        scratch_types=[
            pltpu.VMEM_SHARED((R + NS, D), jnp.float32),
            pltpu.VMEM((CH, D), jnp.float32),
            pltpu.VMEM((CH, D), jnp.float32),
            pltpu.VMEM((1, CH), jnp.int32),
            pltpu.VMEM((1, CH), jnp.int32),
            pltpu.VMEM((1, CH), jnp.int32),
            pltpu.VMEM((1, CH), jnp.int32),
            pltpu.VMEM((CH,), jnp.int32),
            pltpu.VMEM((CH,), jnp.int32),
            pltpu.SemaphoreType.DMA,
            pltpu.SemaphoreType.DMA,
            pltpu.SemaphoreType.DMA,
        ],
    )
    def body(h_hbm, src_hbm, dst_hbm, zer_hbm, out_hbm, acc, rows0, rows1,
             gidx0, gidx1, sidx0, sidx1, dst0, dst1, sem_i, sem_g, sem_s):
        cid = lax.axis_index("c")
        sid = lax.axis_index("s")
        iot = lax.iota(jnp.int32, L)
        rows = (rows0, rows1)
        gidx = (gidx0, gidx1)
        sidxr = (sidx0, sidx1)
        dstage = (dst0, dst1)

        for p in range(npass):
            base = (p * NC + cid) * R
            for c0 in range(0, rpt, 128):
                csz = min(128, rpt - c0)
                pltpu.sync_copy(zer_hbm.at[pl.ds(0, csz)],
                                acc.at[pl.ds(sid * rpt + c0, csz)])
            plsc.subcore_barrier()

            def pair_body(q, _):
                ew = []
                idps = []
                for b in range(2):
                    c = (2 * q + b) * NS + sid
                    e0 = c * CH
                    ewb = jnp.minimum(e0, E - CH)
                    ew.append((ewb, e0))
                    idps.append(pltpu.async_copy(
                        src_hbm.at[pl.ds(ewb, CH)], gidx[b].at[0], sem_i))
                    idps.append(pltpu.async_copy(
                        dst_hbm.at[pl.ds(ewb, CH)], dstage[b], sem_i))
                for d in idps:
                    d.wait()
                gps = [pltpu.async_copy(h_hbm.at[gidx[b].at[0]], rows[b],
                                        sem_g) for b in range(2)]
                for b in range(2):
                    ewb, e0 = ew[b]
                    for j in range(CH // L):
                        dv = dstage[b][pl.ds(j * L, L)]
                        dl = dv - base
                        epos = ewb + j * L + iot
                        ok = (dl >= 0) & (dl < R) & (epos >= e0)
                        sidxr[b][0, pl.ds(j * L, L)] = jnp.where(
                            ok, dl, R + sid)
                sps = []
                for b in range(2):
                    gps[b].wait()
                    sps.append(pltpu.async_copy(
                        rows[b], acc.at[sidxr[b].at[0]], sem_s, add=True))
                for s in sps:
                    s.wait()
                return 0
            lax.fori_loop(0, trips, pair_body, 0)

            plsc.subcore_barrier()
            for c0 in range(0, rpt, 128):
                csz = min(128, rpt - c0)
                lo = sid * rpt + c0
                pltpu.sync_copy(acc.at[pl.ds(lo, csz)],
                                out_hbm.at[pl.ds(p * NC * R + cid * R + lo, csz)])
            if p + 1 < npass:
                plsc.subcore_barrier()

    return body(h, src, dst, zer)


def _pool_sc(h, batch):
    N = h.shape[0]
    G = 256
    assert N % (2 * 8) == 0
    nsc = N // NC
    assert nsc >= CH
    nch = _cdiv(nsc, CH)

    @functools.partial(
        pl.kernel,
        out_type=(jax.ShapeDtypeStruct((NC, G, D), jnp.float32),
                  jax.ShapeDtypeStruct((NC, G, D), jnp.float32)),
        mesh=_mesh(),
        scratch_types=[
            pltpu.VMEM_SHARED((G + 8, D), jnp.float32),
            pltpu.VMEM_SHARED((G + 8, D), jnp.float32),
            pltpu.VMEM((CH, D), jnp.float32),
            pltpu.VMEM((CH,), jnp.int32),
            pltpu.VMEM((1, CH), jnp.int32),
            pltpu.VMEM((CH, D), jnp.float32),
            pltpu.VMEM((16, D), jnp.float32),
            pltpu.SemaphoreType.DMA,
        ],
    )
    def body(h_hbm, b_hbm, sum_hbm, cnt_hbm, accs, accc, rows, bbuf, idxrow,
             obuf, zbuf, sem):
        cid = lax.axis_index("c")
        sid = lax.axis_index("s")
        base = cid * nsc
        one16 = jnp.ones((L,), jnp.float32)
        zero16 = jnp.zeros((L,), jnp.float32)

        def orow(r, _):
            for k in range(D // L):
                obuf[r, pl.ds(k * L, L)] = one16
            return 0
        lax.fori_loop(0, CH, orow, 0)

        def zrow(r, _):
            for k in range(D // L):
                zbuf[r, pl.ds(k * L, L)] = zero16
            return 0
        lax.fori_loop(0, 16, zrow, 0)
        gpt0 = G // NS
        pltpu.sync_copy(zbuf, accs.at[pl.ds(sid * gpt0, gpt0)])
        pltpu.sync_copy(zbuf, accc.at[pl.ds(sid * gpt0, gpt0)])
        plsc.subcore_barrier()

        def cbody(k, _):
            c = sid + k * NS

            @pl.when(c < nch)
            def _():
                ws = jnp.minimum(c * CH, nsc - CH)
                cp = pltpu.async_copy(h_hbm.at[pl.ds(base + ws, CH)], rows, sem)
                pltpu.sync_copy(b_hbm.at[pl.ds(base + ws, CH)], bbuf)
                iot = lax.iota(jnp.int32, L)
                for j in range(CH // L):
                    b_v = bbuf[pl.ds(j * L, L)]
                    gpos = ws + j * L + iot
                    b_v = jnp.where(gpos >= c * CH, b_v, jnp.int32(G))
                    idxrow[0, pl.ds(j * L, L)] = b_v
                cp.wait()
                pltpu.sync_copy(rows, accs.at[idxrow.at[0]], add=True)
                pltpu.sync_copy(obuf, accc.at[idxrow.at[0]], add=True)
            return 0
        lax.fori_loop(0, _cdiv(nch, NS), cbody, 0)
        plsc.subcore_barrier()
        gpt = G // NS
        pltpu.sync_copy(accs.at[pl.ds(sid * gpt, gpt)],
                        sum_hbm.at[cid, pl.ds(sid * gpt, gpt)])
        pltpu.sync_copy(accc.at[pl.ds(sid * gpt, gpt)],
                        cnt_hbm.at[cid, pl.ds(sid * gpt, gpt)])

    return body(h, batch)


BR = 2000


def _mm_relu(xs, ws, bias, n_rows, row_offs=None, gated=None):
    nblk = n_rows // BR
    assert n_rows % BR == 0
    if row_offs is None:
        row_offs = [0] * len(xs)
    if gated is None:
        gated = [None] * len(xs)

    def mk_spec(off, gate):
        def imap(i):
            j = i + off
            if gate is not None:
                j = jnp.minimum(j, gate - 1)
            return (j, 0)
        return pl.BlockSpec((BR, D), imap)

    in_specs = [mk_spec(o, g) for o, g in zip(row_offs, gated)]
    in_specs += [pl.BlockSpec((D, D), lambda i: (0, 0))] * len(ws)
    in_specs += [pl.BlockSpec((1, D), lambda i: (0, 0))]

    def body(*refs):
        i = pl.program_id(0)
        x_refs = refs[:len(xs)]
        w_refs = refs[len(xs):2 * len(xs)]
        b_ref = refs[2 * len(xs)]
        o_ref = refs[2 * len(xs) + 1]
        acc = jnp.dot(x_refs[0][...], w_refs[0][...],
                      preferred_element_type=jnp.float32)
        for t in range(1, len(xs)):
            term = jnp.dot(x_refs[t][...], w_refs[t][...],
                           preferred_element_type=jnp.float32)
            if gated[t] is not None:
                term = jnp.where(i < gated[t], term, 0.0)
            acc = acc + term
        acc = acc + b_ref[...]
        o_ref[...] = jnp.maximum(acc, 0.0)

    return pl.pallas_call(
        body,
        grid=(nblk,),
        in_specs=in_specs,
        out_specs=pl.BlockSpec((BR, D), lambda i: (i, 0)),
        out_shape=jax.ShapeDtypeStruct((n_rows, D), jnp.float32),
    )(*xs, *ws, bias)


def _final_mlp(parts, Wh1, bh1, Wh2, bh2):
    flat = [a for pr in parts for a in pr]

    def body(*refs):
        s0, c0, s1, c1, s2, c2 = refs[:6]
        wh1, b1r, wh2, b2r, o = refs[6:]
        g = jnp.zeros((256, D), jnp.float32)
        for s, c in ((s0, c0), (s1, c1), (s2, c2)):
            ssum = s[0] + s[1]
            cnt = jnp.maximum(c[0] + c[1], 1.0)
            g = g + ssum / cnt
        hid = jnp.maximum(
            jnp.dot(g, wh1[...], preferred_element_type=jnp.float32)
            + b1r[...], 0.0)
        o[...] = (jnp.dot(hid, wh2[...], preferred_element_type=jnp.float32)
                  + b2r[...])

    return pl.pallas_call(
        body,
        in_specs=[pl.BlockSpec((NC, 256, D), lambda: (0, 0, 0))] * 6
        + [pl.BlockSpec((D, D), lambda: (0, 0)),
           pl.BlockSpec((1, D), lambda: (0, 0)),
           pl.BlockSpec((D, 1), lambda: (0, 0)),
           pl.BlockSpec((1, 1), lambda: (0, 0))],
        out_specs=pl.BlockSpec((256, 1), lambda: (0, 0)),
        out_shape=jax.ShapeDtypeStruct((256, 1), jnp.float32),
    )(*flat, Wh1, bh1.reshape(1, D), Wh2, bh2.reshape(1, 1))


def kernel(x0, x1, x2, Ws0, Ws1, Ws2, Wu0, Wu1, Wb1, Wb2, b0, b1, b2, Wh1,
           bh1, Wh2, bh2, boundary_index_1, boundary_index_2,
           upper_adj_index_0, upper_adj_index_1, batch0, batch1, batch2):
    N0, N1, N2 = x0.shape[0], x1.shape[0], x2.shape[0]
    ua0s, ua0d = upper_adj_index_0[0], upper_adj_index_0[1]
    ua1s, ua1d = upper_adj_index_1[0], upper_adj_index_1[1]
    b1s, b1d = boundary_index_1[0], boundary_index_1[1]
    b2s, b2d = boundary_index_2[0], boundary_index_2[1]
    b0r, b1r, b2r = (b.reshape(1, D) for b in (b0, b1, b2))
    nblk0 = N0 // BR
    zer = jnp.zeros((128, D), jnp.float32)

    def layer(h0, h1, h2):
        mu0 = _seg_sum_sc(h0, ua0s, ua0d, zer, N0)
        mb1 = _seg_sum_sc(h0, b1s, b1d, zer, N0)
        mu1 = _seg_sum_sc(h1, ua1s, ua1d, zer, N1)
        mb2 = _seg_sum_sc(h1, b2s, b2d, zer, N2)
        n0 = _mm_relu([h0, mu0], [Ws0, Wu0], b0r, N0)
        n1 = _mm_relu([h1, mu1, mb1], [Ws1, Wu1, Wb1], b1r, N1,
                      gated=[None, None, nblk0])
        n2 = _mm_relu([h2, mb2], [Ws2, Wb2], b2r, N2)
        return n0, n1, n2

    h0, h1, h2 = layer(x0, x1, x2)
    h0, h1, h2 = layer(h0, h1, h2)
    parts = [_pool_sc(h0, batch0), _pool_sc(h1, batch1), _pool_sc(h2, batch2)]
    return _final_mlp(parts, Wh1, bh1, Wh2, bh2)


# speedup vs baseline: 1.6979x; 1.1336x over previous
import functools

import jax
import jax.numpy as jnp
from jax import lax
from jax.experimental import pallas as pl
from jax.experimental.pallas import tpu as pltpu
from jax.experimental.pallas import tpu_sc as plsc

D = 128
NC = 2
NS = 16
L = 16
R = 12544
CH = 112


def _mesh():
    return plsc.VectorSubcoreMesh(
        core_axis_name="c", subcore_axis_name="s", num_cores=NC, num_subcores=NS
    )


def _cdiv(a, b):
    return (a + b - 1) // b


def _seg_sum_sc(h, src, dst, zer, n_out):
    E = src.shape[0]
    npass = _cdiv(n_out, NC * R)
    n_pad = npass * NC * R
    nch = _cdiv(E, CH)
    rpt = R // NS
    trips = _cdiv(_cdiv(nch, NS), 2)

    @functools.partial(
        pl.kernel,
        out_type=jax.ShapeDtypeStruct((n_pad, D), jnp.float32),
        mesh=_mesh(),
        scratch_types=[
            pltpu.VMEM_SHARED((R + NS, D), jnp.float32),
            pltpu.VMEM((CH, D), jnp.float32),
            pltpu.VMEM((CH, D), jnp.float32),
            pltpu.VMEM((1, CH), jnp.int32),
            pltpu.VMEM((1, CH), jnp.int32),
            pltpu.VMEM((1, CH), jnp.int32),
            pltpu.VMEM((1, CH), jnp.int32),
            pltpu.VMEM((CH,), jnp.int32),
            pltpu.VMEM((CH,), jnp.int32),
            pltpu.SemaphoreType.DMA,
            pltpu.SemaphoreType.DMA,
            pltpu.SemaphoreType.DMA,
        ],
    )
    def body(h_hbm, src_hbm, dst_hbm, zer_hbm, out_hbm, acc, rows0, rows1,
             gidx0, gidx1, sidx0, sidx1, dst0, dst1, sem_i, sem_g, sem_s):
        cid = lax.axis_index("c")
        sid = lax.axis_index("s")
        iot = lax.iota(jnp.int32, L)
        rows = (rows0, rows1)
        gidx = (gidx0, gidx1)
        sidxr = (sidx0, sidx1)
        dstage = (dst0, dst1)

        for p in range(npass):
            base = (p * NC + cid) * R
            for c0 in range(0, rpt, 128):
                csz = min(128, rpt - c0)
                pltpu.sync_copy(zer_hbm.at[pl.ds(0, csz)],
                                acc.at[pl.ds(sid * rpt + c0, csz)])
            plsc.subcore_barrier()

            def pair_body(q, _):
                ew = []
                idps = []
                for b in range(2):
                    c = (2 * q + b) * NS + sid
                    e0 = c * CH
                    ewb = jnp.minimum(e0, E - CH)
                    ew.append((ewb, e0))
                    idps.append(pltpu.async_copy(
                        src_hbm.at[pl.ds(ewb, CH)], gidx[b].at[0], sem_i))
                    idps.append(pltpu.async_copy(
                        dst_hbm.at[pl.ds(ewb, CH)], dstage[b], sem_i))

                @pl.when(q > 0)
                def _():
                    for b in range(2):
                        pltpu.make_async_copy(
                            rows[b], acc.at[sidxr[b].at[0]], sem_s).wait()
                for d in idps:
                    d.wait()
                gps = [pltpu.async_copy(h_hbm.at[gidx[b].at[0]], rows[b],
                                        sem_g) for b in range(2)]
                for b in range(2):
                    ewb, e0 = ew[b]
                    for j in range(CH // L):
                        dv = dstage[b][pl.ds(j * L, L)]
                        dl = dv - base
                        epos = ewb + j * L + iot
                        ok = (dl >= 0) & (dl < R) & (epos >= e0)
                        sidxr[b][0, pl.ds(j * L, L)] = jnp.where(
                            ok, dl, R + sid)
                for b in range(2):
                    gps[b].wait()
                    pltpu.async_copy(
                        rows[b], acc.at[sidxr[b].at[0]], sem_s, add=True)
                return 0
            lax.fori_loop(0, trips, pair_body, 0)
            for b in range(2):
                pltpu.make_async_copy(
                    rows[b], acc.at[sidxr[b].at[0]], sem_s).wait()

            plsc.subcore_barrier()
            for c0 in range(0, rpt, 128):
                csz = min(128, rpt - c0)
                lo = sid * rpt + c0
                pltpu.sync_copy(acc.at[pl.ds(lo, csz)],
                                out_hbm.at[pl.ds(p * NC * R + cid * R + lo, csz)])
            if p + 1 < npass:
                plsc.subcore_barrier()

    return body(h, src, dst, zer)


def _pool_sc(h, batch):
    N = h.shape[0]
    G = 256
    assert N % (2 * 8) == 0
    nsc = N // NC
    assert nsc >= CH
    nch = _cdiv(nsc, CH)

    @functools.partial(
        pl.kernel,
        out_type=(jax.ShapeDtypeStruct((NC, G, D), jnp.float32),
                  jax.ShapeDtypeStruct((NC, G, D), jnp.float32)),
        mesh=_mesh(),
        scratch_types=[
            pltpu.VMEM_SHARED((G + 8, D), jnp.float32),
            pltpu.VMEM_SHARED((G + 8, D), jnp.float32),
            pltpu.VMEM((CH, D), jnp.float32),
            pltpu.VMEM((CH,), jnp.int32),
            pltpu.VMEM((1, CH), jnp.int32),
            pltpu.VMEM((CH, D), jnp.float32),
            pltpu.VMEM((16, D), jnp.float32),
            pltpu.SemaphoreType.DMA,
        ],
    )
    def body(h_hbm, b_hbm, sum_hbm, cnt_hbm, accs, accc, rows, bbuf, idxrow,
             obuf, zbuf, sem):
        cid = lax.axis_index("c")
        sid = lax.axis_index("s")
        base = cid * nsc
        one16 = jnp.ones((L,), jnp.float32)
        zero16 = jnp.zeros((L,), jnp.float32)

        def orow(r, _):
            for k in range(D // L):
                obuf[r, pl.ds(k * L, L)] = one16
            return 0
        lax.fori_loop(0, CH, orow, 0)

        def zrow(r, _):
            for k in range(D // L):
                zbuf[r, pl.ds(k * L, L)] = zero16
            return 0
        lax.fori_loop(0, 16, zrow, 0)
        gpt0 = G // NS
        pltpu.sync_copy(zbuf, accs.at[pl.ds(sid * gpt0, gpt0)])
        pltpu.sync_copy(zbuf, accc.at[pl.ds(sid * gpt0, gpt0)])
        plsc.subcore_barrier()

        def cbody(k, _):
            c = sid + k * NS

            @pl.when(c < nch)
            def _():
                ws = jnp.minimum(c * CH, nsc - CH)
                cp = pltpu.async_copy(h_hbm.at[pl.ds(base + ws, CH)], rows, sem)
                pltpu.sync_copy(b_hbm.at[pl.ds(base + ws, CH)], bbuf)
                iot = lax.iota(jnp.int32, L)
                for j in range(CH // L):
                    b_v = bbuf[pl.ds(j * L, L)]
                    gpos = ws + j * L + iot
                    b_v = jnp.where(gpos >= c * CH, b_v, jnp.int32(G))
                    idxrow[0, pl.ds(j * L, L)] = b_v
                cp.wait()
                pltpu.sync_copy(rows, accs.at[idxrow.at[0]], add=True)
                pltpu.sync_copy(obuf, accc.at[idxrow.at[0]], add=True)
            return 0
        lax.fori_loop(0, _cdiv(nch, NS), cbody, 0)
        plsc.subcore_barrier()
        gpt = G // NS
        pltpu.sync_copy(accs.at[pl.ds(sid * gpt, gpt)],
                        sum_hbm.at[cid, pl.ds(sid * gpt, gpt)])
        pltpu.sync_copy(accc.at[pl.ds(sid * gpt, gpt)],
                        cnt_hbm.at[cid, pl.ds(sid * gpt, gpt)])

    return body(h, batch)


BR = 2000


def _mm_relu(xs, ws, bias, n_rows, row_offs=None, gated=None):
    nblk = n_rows // BR
    assert n_rows % BR == 0
    if row_offs is None:
        row_offs = [0] * len(xs)
    if gated is None:
        gated = [None] * len(xs)

    def mk_spec(off, gate):
        def imap(i):
            j = i + off
            if gate is not None:
                j = jnp.minimum(j, gate - 1)
            return (j, 0)
        return pl.BlockSpec((BR, D), imap)

    in_specs = [mk_spec(o, g) for o, g in zip(row_offs, gated)]
    in_specs += [pl.BlockSpec((D, D), lambda i: (0, 0))] * len(ws)
    in_specs += [pl.BlockSpec((1, D), lambda i: (0, 0))]

    def body(*refs):
        i = pl.program_id(0)
        x_refs = refs[:len(xs)]
        w_refs = refs[len(xs):2 * len(xs)]
        b_ref = refs[2 * len(xs)]
        o_ref = refs[2 * len(xs) + 1]
        acc = jnp.dot(x_refs[0][...], w_refs[0][...],
                      preferred_element_type=jnp.float32)
        for t in range(1, len(xs)):
            term = jnp.dot(x_refs[t][...], w_refs[t][...],
                           preferred_element_type=jnp.float32)
            if gated[t] is not None:
                term = jnp.where(i < gated[t], term, 0.0)
            acc = acc + term
        acc = acc + b_ref[...]
        o_ref[...] = jnp.maximum(acc, 0.0)

    return pl.pallas_call(
        body,
        grid=(nblk,),
        in_specs=in_specs,
        out_specs=pl.BlockSpec((BR, D), lambda i: (i, 0)),
        out_shape=jax.ShapeDtypeStruct((n_rows, D), jnp.float32),
    )(*xs, *ws, bias)


def _final_mlp(parts, Wh1, bh1, Wh2, bh2):
    flat = [a for pr in parts for a in pr]

    def body(*refs):
        s0, c0, s1, c1, s2, c2 = refs[:6]
        wh1, b1r, wh2, b2r, o = refs[6:]
        g = jnp.zeros((256, D), jnp.float32)
        for s, c in ((s0, c0), (s1, c1), (s2, c2)):
            ssum = s[0] + s[1]
            cnt = jnp.maximum(c[0] + c[1], 1.0)
            g = g + ssum / cnt
        hid = jnp.maximum(
            jnp.dot(g, wh1[...], preferred_element_type=jnp.float32)
            + b1r[...], 0.0)
        o[...] = (jnp.dot(hid, wh2[...], preferred_element_type=jnp.float32)
                  + b2r[...])

    return pl.pallas_call(
        body,
        in_specs=[pl.BlockSpec((NC, 256, D), lambda: (0, 0, 0))] * 6
        + [pl.BlockSpec((D, D), lambda: (0, 0)),
           pl.BlockSpec((1, D), lambda: (0, 0)),
           pl.BlockSpec((D, 1), lambda: (0, 0)),
           pl.BlockSpec((1, 1), lambda: (0, 0))],
        out_specs=pl.BlockSpec((256, 1), lambda: (0, 0)),
        out_shape=jax.ShapeDtypeStruct((256, 1), jnp.float32),
    )(*flat, Wh1, bh1.reshape(1, D), Wh2, bh2.reshape(1, 1))


def kernel(x0, x1, x2, Ws0, Ws1, Ws2, Wu0, Wu1, Wb1, Wb2, b0, b1, b2, Wh1,
           bh1, Wh2, bh2, boundary_index_1, boundary_index_2,
           upper_adj_index_0, upper_adj_index_1, batch0, batch1, batch2):
    N0, N1, N2 = x0.shape[0], x1.shape[0], x2.shape[0]
    ua0s, ua0d = upper_adj_index_0[0], upper_adj_index_0[1]
    ua1s, ua1d = upper_adj_index_1[0], upper_adj_index_1[1]
    b1s, b1d = boundary_index_1[0], boundary_index_1[1]
    b2s, b2d = boundary_index_2[0], boundary_index_2[1]
    b0r, b1r, b2r = (b.reshape(1, D) for b in (b0, b1, b2))
    nblk0 = N0 // BR
    zer = jnp.zeros((128, D), jnp.float32)

    def layer(h0, h1, h2):
        mu0 = _seg_sum_sc(h0, ua0s, ua0d, zer, N0)
        mb1 = _seg_sum_sc(h0, b1s, b1d, zer, N0)
        mu1 = _seg_sum_sc(h1, ua1s, ua1d, zer, N1)
        mb2 = _seg_sum_sc(h1, b2s, b2d, zer, N2)
        n0 = _mm_relu([h0, mu0], [Ws0, Wu0], b0r, N0)
        n1 = _mm_relu([h1, mu1, mb1], [Ws1, Wu1, Wb1], b1r, N1,
                      gated=[None, None, nblk0])
        n2 = _mm_relu([h2, mb2], [Ws2, Wb2], b2r, N2)
        return n0, n1, n2

    h0, h1, h2 = layer(x0, x1, x2)
    h0, h1, h2 = layer(h0, h1, h2)
    parts = [_pool_sc(h0, batch0), _pool_sc(h1, batch1), _pool_sc(h2, batch2)]
    return _final_mlp(parts, Wh1, bh1, Wh2, bh2)


# speedup vs baseline: 1.7191x; 1.0125x over previous
import functools

import jax
import jax.numpy as jnp
from jax import lax
from jax.experimental import pallas as pl
from jax.experimental.pallas import tpu as pltpu
from jax.experimental.pallas import tpu_sc as plsc

D = 128
NC = 2
NS = 16
L = 16
R = 12544
CH = 112


def _mesh():
    return plsc.VectorSubcoreMesh(
        core_axis_name="c", subcore_axis_name="s", num_cores=NC, num_subcores=NS
    )


def _cdiv(a, b):
    return (a + b - 1) // b


def _seg_sum_sc(h, src, dst, zer, n_out):
    E = src.shape[0]
    npass = _cdiv(n_out, NC * R)
    n_pad = npass * NC * R
    nch = _cdiv(E, CH)
    rpt = R // NS
    trips = _cdiv(_cdiv(nch, NS), 2)

    @functools.partial(
        pl.kernel,
        out_type=jax.ShapeDtypeStruct((n_pad, D), jnp.float32),
        mesh=_mesh(),
        scratch_types=[
            pltpu.VMEM_SHARED((R + NS, D), jnp.float32),
            pltpu.VMEM((CH, D), jnp.float32),
            pltpu.VMEM((CH, D), jnp.float32),
            pltpu.VMEM((2, 2, CH), jnp.int32),
            pltpu.VMEM((1, CH), jnp.int32),
            pltpu.VMEM((1, CH), jnp.int32),
            pltpu.VMEM((2, 2, CH), jnp.int32),
            pltpu.SemaphoreType.DMA,
            pltpu.SemaphoreType.DMA,
            pltpu.SemaphoreType.DMA,
        ],
    )
    def body(h_hbm, src_hbm, dst_hbm, zer_hbm, out_hbm, acc, rows0, rows1,
             gidxb, sidx0, sidx1, dstb, sem_i, sem_g, sem_s):
        cid = lax.axis_index("c")
        sid = lax.axis_index("s")
        iot = lax.iota(jnp.int32, L)
        rows = (rows0, rows1)
        gidx = ((gidxb.at[0, 0], gidxb.at[0, 1]),
                (gidxb.at[1, 0], gidxb.at[1, 1]))
        sidxr = (sidx0, sidx1)
        dstage = ((dstb.at[0, 0], dstb.at[0, 1]),
                  (dstb.at[1, 0], dstb.at[1, 1]))

        for p in range(npass):
            base = (p * NC + cid) * R
            for c0 in range(0, rpt, 128):
                csz = min(128, rpt - c0)
                pltpu.sync_copy(zer_hbm.at[pl.ds(0, csz)],
                                acc.at[pl.ds(sid * rpt + c0, csz)])
            plsc.subcore_barrier()

            def issue_idx(s, qq):
                for b in range(2):
                    c = (2 * qq + b) * NS + sid
                    ewb = jnp.minimum(c * CH, E - CH)
                    pltpu.async_copy(src_hbm.at[pl.ds(ewb, CH)],
                                     gidx[s][b], sem_i)
                    pltpu.async_copy(dst_hbm.at[pl.ds(ewb, CH)],
                                     dstage[s][b], sem_i)

            def wait_idx(s):
                for b in range(2):
                    pltpu.make_async_copy(src_hbm.at[pl.ds(0, CH)],
                                          gidx[s][b], sem_i).wait()
                    pltpu.make_async_copy(dst_hbm.at[pl.ds(0, CH)],
                                          dstage[s][b], sem_i).wait()

            def drain_scatters():
                for b in range(2):
                    pltpu.make_async_copy(
                        rows[b], acc.at[sidxr[b].at[0]], sem_s).wait()

            def run_pair(s, qq, drain_cond):
                wait_idx(s)
                issue_idx(1 - s, qq + 1)
                if drain_cond is None:
                    drain_scatters()
                else:
                    @pl.when(drain_cond)
                    def _():
                        drain_scatters()
                gps = [pltpu.async_copy(h_hbm.at[gidx[s][b]], rows[b],
                                        sem_g) for b in range(2)]
                for b in range(2):
                    c = (2 * qq + b) * NS + sid
                    e0 = c * CH
                    ewb = jnp.minimum(e0, E - CH)
                    for j in range(CH // L):
                        dv = dstage[s][b][pl.ds(j * L, L)]
                        dl = dv - base
                        epos = ewb + j * L + iot
                        ok = (dl >= 0) & (dl < R) & (epos >= e0)
                        sidxr[b][0, pl.ds(j * L, L)] = jnp.where(
                            ok, dl, R + sid)
                for b in range(2):
                    gps[b].wait()
                    pltpu.async_copy(
                        rows[b], acc.at[sidxr[b].at[0]], sem_s, add=True)

            issue_idx(0, jnp.int32(0))
            trips2 = _cdiv(trips, 2)

            def duo_body(q2, _):
                qq = 2 * q2
                run_pair(0, qq, q2 > 0)
                run_pair(1, qq + 1, None)
                return 0
            lax.fori_loop(0, trips2, duo_body, 0)
            wait_idx(0)
            drain_scatters()

            plsc.subcore_barrier()
            for c0 in range(0, rpt, 128):
                csz = min(128, rpt - c0)
                lo = sid * rpt + c0
                pltpu.sync_copy(acc.at[pl.ds(lo, csz)],
                                out_hbm.at[pl.ds(p * NC * R + cid * R + lo, csz)])
            if p + 1 < npass:
                plsc.subcore_barrier()

    return body(h, src, dst, zer)


def _pool_sc(h, batch):
    N = h.shape[0]
    G = 256
    assert N % (2 * 8) == 0
    nsc = N // NC
    assert nsc >= CH
    nch = _cdiv(nsc, CH)

    @functools.partial(
        pl.kernel,
        out_type=(jax.ShapeDtypeStruct((NC, G, D), jnp.float32),
                  jax.ShapeDtypeStruct((NC, G, D), jnp.float32)),
        mesh=_mesh(),
        scratch_types=[
            pltpu.VMEM_SHARED((G + 8, D), jnp.float32),
            pltpu.VMEM_SHARED((G + 8, D), jnp.float32),
            pltpu.VMEM((CH, D), jnp.float32),
            pltpu.VMEM((CH,), jnp.int32),
            pltpu.VMEM((1, CH), jnp.int32),
            pltpu.VMEM((CH, D), jnp.float32),
            pltpu.VMEM((16, D), jnp.float32),
            pltpu.SemaphoreType.DMA,
        ],
    )
    def body(h_hbm, b_hbm, sum_hbm, cnt_hbm, accs, accc, rows, bbuf, idxrow,
             obuf, zbuf, sem):
        cid = lax.axis_index("c")
        sid = lax.axis_index("s")
        base = cid * nsc
        one16 = jnp.ones((L,), jnp.float32)
        zero16 = jnp.zeros((L,), jnp.float32)

        def orow(r, _):
            for k in range(D // L):
                obuf[r, pl.ds(k * L, L)] = one16
            return 0
        lax.fori_loop(0, CH, orow, 0)

        def zrow(r, _):
            for k in range(D // L):
                zbuf[r, pl.ds(k * L, L)] = zero16
            return 0
        lax.fori_loop(0, 16, zrow, 0)
        gpt0 = G // NS
        pltpu.sync_copy(zbuf, accs.at[pl.ds(sid * gpt0, gpt0)])
        pltpu.sync_copy(zbuf, accc.at[pl.ds(sid * gpt0, gpt0)])
        plsc.subcore_barrier()

        def cbody(k, _):
            c = sid + k * NS

            @pl.when(c < nch)
            def _():
                ws = jnp.minimum(c * CH, nsc - CH)
                cp = pltpu.async_copy(h_hbm.at[pl.ds(base + ws, CH)], rows, sem)
                pltpu.sync_copy(b_hbm.at[pl.ds(base + ws, CH)], bbuf)
                iot = lax.iota(jnp.int32, L)
                for j in range(CH // L):
                    b_v = bbuf[pl.ds(j * L, L)]
                    gpos = ws + j * L + iot
                    b_v = jnp.where(gpos >= c * CH, b_v, jnp.int32(G))
                    idxrow[0, pl.ds(j * L, L)] = b_v
                cp.wait()
                pltpu.sync_copy(rows, accs.at[idxrow.at[0]], add=True)
                pltpu.sync_copy(obuf, accc.at[idxrow.at[0]], add=True)
            return 0
        lax.fori_loop(0, _cdiv(nch, NS), cbody, 0)
        plsc.subcore_barrier()
        gpt = G // NS
        pltpu.sync_copy(accs.at[pl.ds(sid * gpt, gpt)],
                        sum_hbm.at[cid, pl.ds(sid * gpt, gpt)])
        pltpu.sync_copy(accc.at[pl.ds(sid * gpt, gpt)],
                        cnt_hbm.at[cid, pl.ds(sid * gpt, gpt)])

    return body(h, batch)


BR = 2000


def _mm_relu(xs, ws, bias, n_rows, row_offs=None, gated=None):
    nblk = n_rows // BR
    assert n_rows % BR == 0
    if row_offs is None:
        row_offs = [0] * len(xs)
    if gated is None:
        gated = [None] * len(xs)

    def mk_spec(off, gate):
        def imap(i):
            j = i + off
            if gate is not None:
                j = jnp.minimum(j, gate - 1)
            return (j, 0)
        return pl.BlockSpec((BR, D), imap)

    in_specs = [mk_spec(o, g) for o, g in zip(row_offs, gated)]
    in_specs += [pl.BlockSpec((D, D), lambda i: (0, 0))] * len(ws)
    in_specs += [pl.BlockSpec((1, D), lambda i: (0, 0))]

    def body(*refs):
        i = pl.program_id(0)
        x_refs = refs[:len(xs)]
        w_refs = refs[len(xs):2 * len(xs)]
        b_ref = refs[2 * len(xs)]
        o_ref = refs[2 * len(xs) + 1]
        acc = jnp.dot(x_refs[0][...], w_refs[0][...],
                      preferred_element_type=jnp.float32)
        for t in range(1, len(xs)):
            term = jnp.dot(x_refs[t][...], w_refs[t][...],
                           preferred_element_type=jnp.float32)
            if gated[t] is not None:
                term = jnp.where(i < gated[t], term, 0.0)
            acc = acc + term
        acc = acc + b_ref[...]
        o_ref[...] = jnp.maximum(acc, 0.0)

    return pl.pallas_call(
        body,
        grid=(nblk,),
        in_specs=in_specs,
        out_specs=pl.BlockSpec((BR, D), lambda i: (i, 0)),
        out_shape=jax.ShapeDtypeStruct((n_rows, D), jnp.float32),
    )(*xs, *ws, bias)


def _final_mlp(parts, Wh1, bh1, Wh2, bh2):
    flat = [a for pr in parts for a in pr]

    def body(*refs):
        s0, c0, s1, c1, s2, c2 = refs[:6]
        wh1, b1r, wh2, b2r, o = refs[6:]
        g = jnp.zeros((256, D), jnp.float32)
        for s, c in ((s0, c0), (s1, c1), (s2, c2)):
            ssum = s[0] + s[1]
            cnt = jnp.maximum(c[0] + c[1], 1.0)
            g = g + ssum / cnt
        hid = jnp.maximum(
            jnp.dot(g, wh1[...], preferred_element_type=jnp.float32)
            + b1r[...], 0.0)
        o[...] = (jnp.dot(hid, wh2[...], preferred_element_type=jnp.float32)
                  + b2r[...])

    return pl.pallas_call(
        body,
        in_specs=[pl.BlockSpec((NC, 256, D), lambda: (0, 0, 0))] * 6
        + [pl.BlockSpec((D, D), lambda: (0, 0)),
           pl.BlockSpec((1, D), lambda: (0, 0)),
           pl.BlockSpec((D, 1), lambda: (0, 0)),
           pl.BlockSpec((1, 1), lambda: (0, 0))],
        out_specs=pl.BlockSpec((256, 1), lambda: (0, 0)),
        out_shape=jax.ShapeDtypeStruct((256, 1), jnp.float32),
    )(*flat, Wh1, bh1.reshape(1, D), Wh2, bh2.reshape(1, 1))


def kernel(x0, x1, x2, Ws0, Ws1, Ws2, Wu0, Wu1, Wb1, Wb2, b0, b1, b2, Wh1,
           bh1, Wh2, bh2, boundary_index_1, boundary_index_2,
           upper_adj_index_0, upper_adj_index_1, batch0, batch1, batch2):
    N0, N1, N2 = x0.shape[0], x1.shape[0], x2.shape[0]
    ua0s, ua0d = upper_adj_index_0[0], upper_adj_index_0[1]
    ua1s, ua1d = upper_adj_index_1[0], upper_adj_index_1[1]
    b1s, b1d = boundary_index_1[0], boundary_index_1[1]
    b2s, b2d = boundary_index_2[0], boundary_index_2[1]
    b0r, b1r, b2r = (b.reshape(1, D) for b in (b0, b1, b2))
    nblk0 = N0 // BR
    zer = jnp.zeros((128, D), jnp.float32)

    def layer(h0, h1, h2):
        mu0 = _seg_sum_sc(h0, ua0s, ua0d, zer, N0)
        mb1 = _seg_sum_sc(h0, b1s, b1d, zer, N0)
        mu1 = _seg_sum_sc(h1, ua1s, ua1d, zer, N1)
        mb2 = _seg_sum_sc(h1, b2s, b2d, zer, N2)
        n0 = _mm_relu([h0, mu0], [Ws0, Wu0], b0r, N0)
        n1 = _mm_relu([h1, mu1, mb1], [Ws1, Wu1, Wb1], b1r, N1,
                      gated=[None, None, nblk0])
        n2 = _mm_relu([h2, mb2], [Ws2, Wb2], b2r, N2)
        return n0, n1, n2

    h0, h1, h2 = layer(x0, x1, x2)
    h0, h1, h2 = layer(h0, h1, h2)
    parts = [_pool_sc(h0, batch0), _pool_sc(h1, batch1), _pool_sc(h2, batch2)]
    return _final_mlp(parts, Wh1, bh1, Wh2, bh2)
